# baseline (speedup 1.0000x reference)
import functools

import jax
import jax.numpy as jnp
from jax import lax
from jax.experimental import pallas as pl
from jax.experimental.pallas import tpu as pltpu
from jax.experimental.pallas import tpu_sc as plsc

_LANES = 16


def _build_gather(batch, dim, num_heads, s_rows, pack):
    info = plsc.get_sparse_core_info()
    nc, ns = info.num_cores, info.num_subcores
    assert num_heads % nc == 0
    h_per_w = num_heads // nc
    b_per_w = batch // ns
    assert b_per_w * ns == batch and b_per_w % 8 == 0
    total = batch * num_heads
    vregs_per_row = b_per_w // _LANES
    assert vregs_per_row * _LANES == b_per_w

    mesh = plsc.VectorSubcoreMesh(core_axis_name="c", subcore_axis_name="s")

    @functools.partial(
        pl.kernel,
        mesh=mesh,
        out_type=jax.ShapeDtypeStruct((total, 128), jnp.float32),
        compiler_params=pltpu.CompilerParams(use_tc_tiling_on_sc=False),
        scratch_types=[
            pltpu.VMEM((num_heads * _LANES,), jnp.int32),
            pltpu.VMEM((num_heads * _LANES,), jnp.int32),
            pltpu.VMEM((h_per_w, b_per_w), jnp.int32),
            pltpu.VMEM((b_per_w, dim), jnp.float32),
            pltpu.VMEM((b_per_w, dim), jnp.float32),
            pltpu.SemaphoreType.DMA,
            pltpu.SemaphoreType.DMA,
            pltpu.SemaphoreType.DMA,
            pltpu.SemaphoreType.DMA,
        ],
    )
    def gather_kernel(hash_t_hbm, a_hbm, c_hbm, table_hbm, out_hbm,
                      a_v, c_v, ids_v, rows0, rows1,
                      gsem0, gsem1, wsem0, wsem1):
        wh = lax.axis_index("c")
        wb = lax.axis_index("s")
        h0 = wh * h_per_w
        b0 = wb * b_per_w

        pltpu.sync_copy(hash_t_hbm.at[pl.ds(h0, h_per_w), pl.ds(b0, b_per_w)],
                        ids_v)
        pltpu.sync_copy(a_hbm, a_v)
        pltpu.sync_copy(c_hbm, c_v)

        pat_a = [a_v[pl.ds((h0 + hl) * _LANES, _LANES)]
                 for hl in range(h_per_w)]
        pat_c = [c_v[pl.ds((h0 + hl) * _LANES, _LANES)]
                 for hl in range(h_per_w)]
        wrap = jnp.full((_LANES,), pack * s_rows - 1, jnp.int32)

        def add_body(j, carry):
            s = j * _LANES
            for hl in range(h_per_w):
                hsh = ids_v[hl, pl.ds(s, _LANES)]
                idx = hsh * pack + pat_a[hl]
                ids_v[hl, pl.ds(s, _LANES)] = jnp.where(
                    hsh >= pat_c[hl], idx - wrap, idx)
            return carry

        lax.fori_loop(0, vregs_per_row, add_body, 0)

        rows = (rows0, rows1)
        gsems = (gsem0, gsem1)
        wsems = (wsem0, wsem1)
        for hl in range(h_per_w):
            j = hl % 2
            if hl >= 2:
                pltpu.make_async_copy(
                    rows[j],
                    out_hbm.at[pl.ds((h0 + hl - 2) * batch + b0, b_per_w),
                               pl.ds(0, dim)],
                    wsems[j]).wait()
            pltpu.async_copy(
                table_hbm.at[ids_v.at[hl]], rows[j], gsems[j]).wait()
            pltpu.async_copy(
                rows[j],
                out_hbm.at[pl.ds((h0 + hl) * batch + b0, b_per_w),
                           pl.ds(0, dim)],
                wsems[j])
        for hl in (h_per_w - 2, h_per_w - 1):
            j = hl % 2
            pltpu.make_async_copy(
                rows[j],
                out_hbm.at[pl.ds((h0 + hl) * batch + b0, b_per_w),
                           pl.ds(0, dim)],
                wsems[j]).wait()

    return gather_kernel


def _tc_detranspose(table_t, dim, col_block, n_grid):
    pack = 128 // dim
    s_rows = n_grid * col_block

    def body(*refs):
        xs, o_ref = refs[:-1], refs[-1]
        stacked = jnp.concatenate([x[...] for x in xs], axis=0)
        o_ref[...] = jnp.swapaxes(stacked, 0, 1)

    max_blk = (table_t.shape[1] - 1) // col_block
    specs = [
        pl.BlockSpec(
            (dim, col_block),
            functools.partial(
                lambda k, j: (0, jnp.minimum(j + k * n_grid, max_blk)), k))
        for k in range(pack)
    ]
    return pl.pallas_call(
        body,
        grid=(n_grid,),
        in_specs=specs,
        out_specs=pl.BlockSpec((col_block, 128), lambda j: (j, 0)),
        out_shape=jax.ShapeDtypeStruct((s_rows, 128), jnp.float32),
    )(*([table_t] * pack))


def kernel(hash_ids, table, offsets):
    batch, num_heads = hash_ids.shape
    table_rows, dim = table.shape
    hash_t = hash_ids.T
    pack = 128 // dim
    col_block = 16384
    n_grid = -(-table_rows // (pack * col_block))
    s_rows = n_grid * col_block
    table_c = _tc_detranspose(table.T, dim, col_block, n_grid)
    table_c = table_c.reshape(pack * s_rows, dim)
    k0 = offsets // s_rows
    a_pat = jnp.repeat(pack * offsets - (pack * s_rows - 1) * k0, _LANES)
    c_pat = jnp.repeat(s_rows * (k0 + 1) - offsets, _LANES)
    gk = _build_gather(batch, dim, num_heads, s_rows, pack)
    out_t = gk(hash_t, a_pat, c_pat, table_c)
    out_t = out_t[:, :dim].reshape(num_heads, batch, dim)
    return out_t.transpose(1, 0, 2)
